# baseline (speedup 1.0000x reference)
import functools

import jax
import jax.numpy as jnp
from jax import lax
from jax.experimental import pallas as pl
from jax.experimental.pallas import tpu as pltpu
from jax.experimental.pallas import tpu_sc as plsc

_N = 10000
_E = 320000
_DIN = 128
_DE = 16
_DOUT = 128

_NC, _NS, _L = 2, 16, 16
_EPT = _E // (_NC * _NS)
_EPC = _E // _NC
_SUB = 2000
_NSUB = _EPT // _SUB
_NP = 10240
_RPT = _NP // _NS

_SC_PARAMS = pltpu.CompilerParams(needs_layout_passes=False,
                                  use_tc_tiling_on_sc=False)
_MESH = plsc.VectorSubcoreMesh(core_axis_name="c", subcore_axis_name="s",
                               num_cores=_NC, num_subcores=_NS)


def _sc_denom(sp8_hbm, ei_hbm, z1d_hbm, eexp_hbm, dpart_hbm, dstl_hbm,
              ssrc_v, sdst_v, src_v, dst_v, eexp_v, denom_sh):
  c = lax.axis_index("c")
  s = lax.axis_index("s")
  r0 = s * _RPT
  ofs = c * _EPC + s * _EPT

  pltpu.sync_copy(z1d_hbm.at[pl.ds(r0, _RPT)], denom_sh.at[pl.ds(r0, _RPT)])
  pltpu.sync_copy(sp8_hbm.at[0], ssrc_v)
  pltpu.sync_copy(sp8_hbm.at[1], sdst_v)
  pltpu.sync_copy(ei_hbm.at[0, pl.ds(ofs, _EPT)], src_v)
  pltpu.sync_copy(ei_hbm.at[1, pl.ds(ofs, _EPT)], dst_v)
  plsc.subcore_barrier()

  def body(i, carry):
    sl = pl.ds(i * _L, _L)
    vs = plsc.load_gather(ssrc_v, [src_v[sl]])
    vd = plsc.load_gather(sdst_v, [dst_v[sl]])
    eexp_v[sl] = jnp.exp(jnp.maximum(vs + vd, 0.0))
    return carry
  lax.fori_loop(0, _EPT // _L, body, 0)

  pltpu.sync_copy(eexp_v, eexp_hbm.at[pl.ds(ofs, _EPT)])
  pltpu.sync_copy(dst_v, dstl_hbm.at[pl.ds(ofs, _EPT)])
  pltpu.sync_copy(eexp_v, denom_sh.at[dst_v], add=True)
  plsc.subcore_barrier()
  pltpu.sync_copy(denom_sh.at[pl.ds(r0, _RPT)], dpart_hbm.at[c, pl.ds(r0, _RPT)])


_sc_denom_call = functools.partial(
    pl.kernel,
    out_type=(jax.ShapeDtypeStruct((_E,), jnp.float32),
              jax.ShapeDtypeStruct((_NC, _NP), jnp.float32),
              jax.ShapeDtypeStruct((_E,), jnp.int32)),
    mesh=_MESH,
    compiler_params=_SC_PARAMS,
    scratch_types=[
        pltpu.VMEM((_N,), jnp.float32),
        pltpu.VMEM((_N,), jnp.float32),
        pltpu.VMEM((_EPT,), jnp.int32),
        pltpu.VMEM((_EPT,), jnp.int32),
        pltpu.VMEM((_EPT,), jnp.float32),
        pltpu.VMEM_SHARED((_NP,), jnp.float32),
    ],
)(_sc_denom)


_SPLIT = 159744
_RMAX = 1264


def _chunks(cnt, k=4):
  g = cnt // 16
  out = []
  for i in range(k):
    gi = g // k + (1 if i < g % k else 0)
    out.append(gi * 16)
  return out


def _make_zsum(base, total):
  cnt = total // (_NC * _NS)
  epc = total // _NC
  sizes = _chunks(cnt)

  def _sc_zsum(dstl_hbm, eexp_hbm, ef_hbm, zinit_hbm,
               zpart_hbm,
               dst_v, eexp_v, raw0_v, raw1_v, sc0_v, sc1_v, z_sh, lsem, ssem):
    c = lax.axis_index("c")
    s = lax.axis_index("s")
    r0 = s * _RPT
    ofs_l = c * epc + s * cnt
    ofs_g = base + ofs_l
    raws = (raw0_v, raw1_v)
    sbufs = (sc0_v, sc1_v)

    zsrc = zinit_hbm if base == 0 else zinit_hbm.at[c]
    cz = pltpu.async_copy(zsrc.at[pl.ds(r0, _RPT)],
                          z_sh.at[pl.ds(r0, _RPT)], lsem)
    cd = pltpu.async_copy(dstl_hbm.at[pl.ds(ofs_g, cnt)], dst_v, lsem)
    ce = pltpu.async_copy(eexp_hbm.at[pl.ds(ofs_g, cnt)], eexp_v, lsem)
    loads = [pltpu.async_copy(ef_hbm.at[pl.ds(ofs_l // 8, sizes[0] // 8)],
                              raw0_v.at[pl.ds(0, sizes[0] // 8)], lsem)]
    cz.wait()
    cd.wait()
    ce.wait()
    plsc.subcore_barrier()

    scatters = []
    off = 0
    for j, sz in enumerate(sizes):
      raw = raws[j % 2]
      sbuf = sbufs[j % 2]
      loads[j].wait()
      if j + 1 < len(sizes):
        nsz = sizes[j + 1]
        loads.append(pltpu.async_copy(
            ef_hbm.at[pl.ds((ofs_l + off + sz) // 8, nsz // 8)],
            raws[(j + 1) % 2].at[pl.ds(0, nsz // 8)], lsem))

      def sbody(g, carry, off=off, raw=raw, sbuf=sbuf):
        a = eexp_v[pl.ds(off + g * _L, _L)]
        for k in range(_L):
          e = g * _L + k
          r = g * 2 + (k // 8)
          sbuf[e, :] = raw[r, pl.ds((k % 8) * _L, _L)] * a[k]
        return carry
      lax.fori_loop(0, sz // _L, sbody, 0)

      if j >= 1:
        scatters[j - 1].wait()
      scatters.append(pltpu.async_copy(
          sbuf.at[pl.ds(0, sz)], z_sh.at[dst_v.at[pl.ds(off, sz)]], ssem,
          add=True))
      off += sz

    scatters[-1].wait()
    plsc.subcore_barrier()
    pltpu.sync_copy(z_sh.at[pl.ds(r0, _RPT)], zpart_hbm.at[c, pl.ds(r0, _RPT)])

  return functools.partial(
      pl.kernel,
      out_type=jax.ShapeDtypeStruct((_NC, _NP, _DE), jnp.float32),
      mesh=_MESH,
      compiler_params=_SC_PARAMS,
      scratch_types=[
          pltpu.VMEM((cnt,), jnp.int32),
          pltpu.VMEM((cnt,), jnp.float32),
          pltpu.VMEM((_RMAX // 8, 128), jnp.float32),
          pltpu.VMEM((_RMAX // 8, 128), jnp.float32),
          pltpu.VMEM((_RMAX, _DE), jnp.float32),
          pltpu.VMEM((_RMAX, _DE), jnp.float32),
          pltpu.VMEM_SHARED((_NP, _DE), jnp.float32),
          pltpu.SemaphoreType.DMA,
          pltpu.SemaphoreType.DMA,
      ],
  )(_sc_zsum)


_sc_zsum_calls = (_make_zsum(0, _SPLIT), _make_zsum(_SPLIT, _E - _SPLIT))


def _s_tc(x_ref, a_ref, b_ref, o_ref):
  o_ref[...] = lax.dot_general(a_ref[...], x_ref[...], (((1,), (1,)), ((), ())),
                               preferred_element_type=jnp.float32) + b_ref[...]


def _out_tc(x_ref, za_ref, zb_ref, d0_ref, d1_ref, w_ref, b_ref, o_ref):
  nb = x_ref.shape[0]
  d = d0_ref[...] + d1_ref[...]
  z = (za_ref[...] + zb_ref[...]) * jnp.where(d > 0.0, 1.0 / d, 0.0)
  dn = (((1,), (1,)), ((), ()))
  acc = lax.dot_general(x_ref[...], w_ref[:, :_DIN], dn,
                        preferred_element_type=jnp.float32)
  acc += lax.dot_general(z, w_ref[:, _DIN:], dn,
                         preferred_element_type=jnp.float32)
  o_ref[...] = jnp.maximum(acc + b_ref[...], 0.0).reshape(nb, 1, _DOUT)


def kernel(nfeats, efeats, edge_index, W_w, W_b, A_w, A_b):
  ei = edge_index.astype(jnp.int32)
  ef0 = efeats[:_SPLIT].reshape(_SPLIT // 8, 128).astype(jnp.float32)
  ef1 = efeats[_SPLIT:].reshape((_E - _SPLIT) // 8, 128).astype(jnp.float32)
  X2d = nfeats.reshape(_N, _DIN)

  A8 = jnp.zeros((8, _DIN), jnp.float32)
  A8 = A8.at[0].set(A_w[0, :_DIN]).at[1].set(A_w[0, _DIN:])
  sp8 = pl.pallas_call(
      _s_tc,
      out_shape=jax.ShapeDtypeStruct((8, _N), jnp.float32),
  )(X2d, A8, jnp.zeros((8, 1), jnp.float32).at[1, 0].set(A_b[0]))

  eexp, dpart, dstl = _sc_denom_call(sp8, ei, jnp.zeros((_NP,), jnp.float32))
  zpart0 = _sc_zsum_calls[0](dstl, eexp, ef0,
                             jnp.zeros((_NP, _DE), jnp.float32))
  zpart = _sc_zsum_calls[1](dstl, eexp, ef1, zpart0)

  out = pl.pallas_call(
      _out_tc,
      grid=(10,),
      in_specs=[
          pl.BlockSpec((_N // 10, _DIN), lambda i: (i, 0)),
          pl.BlockSpec((_N // 10, _DE), lambda i: (i, 0)),
          pl.BlockSpec((_N // 10, _DE), lambda i: (i, 0)),
          pl.BlockSpec((_N // 10, 1), lambda i: (i, 0)),
          pl.BlockSpec((_N // 10, 1), lambda i: (i, 0)),
          pl.BlockSpec((_DOUT, _DIN + _DE), lambda i: (0, 0)),
          pl.BlockSpec((1, _DOUT), lambda i: (0, 0)),
      ],
      out_specs=pl.BlockSpec((_N // 10, 1, _DOUT), lambda i: (i, 0, 0)),
      out_shape=jax.ShapeDtypeStruct((_N, 1, _DOUT), jnp.float32),
  )(X2d, zpart[0, :_N], zpart[1, :_N],
    dpart[0, :_N].reshape(_N, 1), dpart[1, :_N].reshape(_N, 1),
    W_w, W_b.reshape(1, _DOUT))
  return out


# speedup vs baseline: 1.0178x; 1.0178x over previous
import functools

import jax
import jax.numpy as jnp
from jax import lax
from jax.experimental import pallas as pl
from jax.experimental.pallas import tpu as pltpu
from jax.experimental.pallas import tpu_sc as plsc

_N = 10000
_E = 320000
_DIN = 128
_DE = 16
_DOUT = 128

_NC, _NS, _L = 2, 16, 16
_EPT = _E // (_NC * _NS)
_EPC = _E // _NC
_SUB = 2000
_NSUB = _EPT // _SUB
_NP = 10240
_RPT = _NP // _NS

_SC_PARAMS = pltpu.CompilerParams(needs_layout_passes=False,
                                  use_tc_tiling_on_sc=False)
_MESH = plsc.VectorSubcoreMesh(core_axis_name="c", subcore_axis_name="s",
                               num_cores=_NC, num_subcores=_NS)


def _sc_denom(sp8_hbm, ei_hbm, eexp_hbm, dpart_hbm, dstl_hbm,
              ssrc_v, sdst_v, src_v, dst_v, eexp_v, zb_v, denom_sh):
  c = lax.axis_index("c")
  s = lax.axis_index("s")
  r0 = s * _RPT
  ofs = c * _EPC + s * _EPT

  def zbody(i, carry):
    zb_v[pl.ds(i * _L, _L)] = jnp.zeros((_L,), jnp.float32)
    return carry
  lax.fori_loop(0, _RPT // _L, zbody, 0)
  pltpu.sync_copy(zb_v, denom_sh.at[pl.ds(r0, _RPT)])
  pltpu.sync_copy(sp8_hbm.at[0], ssrc_v)
  pltpu.sync_copy(sp8_hbm.at[1], sdst_v)
  pltpu.sync_copy(ei_hbm.at[0, pl.ds(ofs, _EPT)], src_v)
  pltpu.sync_copy(ei_hbm.at[1, pl.ds(ofs, _EPT)], dst_v)
  plsc.subcore_barrier()

  def body(i, carry):
    sl = pl.ds(i * _L, _L)
    vs = plsc.load_gather(ssrc_v, [src_v[sl]])
    vd = plsc.load_gather(sdst_v, [dst_v[sl]])
    eexp_v[sl] = jnp.exp(jnp.maximum(vs + vd, 0.0))
    return carry
  lax.fori_loop(0, _EPT // _L, body, 0)

  pltpu.sync_copy(eexp_v, eexp_hbm.at[pl.ds(ofs, _EPT)])
  pltpu.sync_copy(dst_v, dstl_hbm.at[pl.ds(ofs, _EPT)])
  pltpu.sync_copy(eexp_v, denom_sh.at[dst_v], add=True)
  plsc.subcore_barrier()
  pltpu.sync_copy(denom_sh.at[pl.ds(r0, _RPT)], dpart_hbm.at[c, pl.ds(r0, _RPT)])


_sc_denom_call = functools.partial(
    pl.kernel,
    out_type=(jax.ShapeDtypeStruct((_E,), jnp.float32),
              jax.ShapeDtypeStruct((_NC, _NP), jnp.float32),
              jax.ShapeDtypeStruct((_E,), jnp.int32)),
    mesh=_MESH,
    compiler_params=_SC_PARAMS,
    scratch_types=[
        pltpu.VMEM((_N,), jnp.float32),
        pltpu.VMEM((_N,), jnp.float32),
        pltpu.VMEM((_EPT,), jnp.int32),
        pltpu.VMEM((_EPT,), jnp.int32),
        pltpu.VMEM((_EPT,), jnp.float32),
        pltpu.VMEM((_RPT,), jnp.float32),
        pltpu.VMEM_SHARED((_NP,), jnp.float32),
    ],
)(_sc_denom)


_SPLIT = 159744
_RMAX = 1264


def _chunks(cnt, k=4):
  g = cnt // 16
  out = []
  for i in range(k):
    gi = g // k + (1 if i < g % k else 0)
    out.append(gi * 16)
  return out


def _make_zsum(base, total):
  cnt = total // (_NC * _NS)
  epc = total // _NC
  sizes = _chunks(cnt)

  def _sc_zsum(dstl_hbm, eexp_hbm, ef_hbm, *args):
    if base == 0:
      zpart_hbm = args[0]
      (dst_v, eexp_v, raw0_v, raw1_v, sc0_v, sc1_v, zb_v, d0_v, d1_v, z_sh,
       lsem, ssem) = args[1:]
    else:
      zinit_hbm, dpart_hbm, zpart_hbm = args[0], args[1], args[2]
      (dst_v, eexp_v, raw0_v, raw1_v, sc0_v, sc1_v, zb_v, d0_v, d1_v, z_sh,
       lsem, ssem) = args[3:]
    c = lax.axis_index("c")
    s = lax.axis_index("s")
    r0 = s * _RPT
    ofs_l = c * epc + s * cnt
    ofs_g = base + ofs_l
    raws = (raw0_v, raw1_v)
    sbufs = (sc0_v, sc1_v)

    cd = pltpu.async_copy(dstl_hbm.at[pl.ds(ofs_g, cnt)], dst_v, lsem)
    ce = pltpu.async_copy(eexp_hbm.at[pl.ds(ofs_g, cnt)], eexp_v, lsem)
    loads = [pltpu.async_copy(ef_hbm.at[pl.ds(ofs_l // 8, sizes[0] // 8)],
                              raw0_v.at[pl.ds(0, sizes[0] // 8)], lsem)]
    if base == 0:
      def zbody(i, carry):
        zb_v[i, :] = jnp.zeros((_L,), jnp.float32)
        return carry
      lax.fori_loop(0, _RPT, zbody, 0)
      pltpu.sync_copy(zb_v, z_sh.at[pl.ds(r0, _RPT)])
    else:
      cz = pltpu.async_copy(zinit_hbm.at[c, pl.ds(r0, _RPT)],
                            z_sh.at[pl.ds(r0, _RPT)], lsem)
      c0 = pltpu.async_copy(dpart_hbm.at[0, pl.ds(r0, _RPT)], d0_v, lsem)
      c1 = pltpu.async_copy(dpart_hbm.at[1, pl.ds(r0, _RPT)], d1_v, lsem)
      cz.wait()
      c0.wait()
      c1.wait()
    cd.wait()
    ce.wait()
    plsc.subcore_barrier()

    scatters = []
    off = 0
    for j, sz in enumerate(sizes):
      raw = raws[j % 2]
      sbuf = sbufs[j % 2]
      loads[j].wait()
      if j + 1 < len(sizes):
        nsz = sizes[j + 1]
        loads.append(pltpu.async_copy(
            ef_hbm.at[pl.ds((ofs_l + off + sz) // 8, nsz // 8)],
            raws[(j + 1) % 2].at[pl.ds(0, nsz // 8)], lsem))

      def sbody(g, carry, off=off, raw=raw, sbuf=sbuf):
        a = eexp_v[pl.ds(off + g * _L, _L)]
        for k in range(_L):
          e = g * _L + k
          r = g * 2 + (k // 8)
          sbuf[e, :] = raw[r, pl.ds((k % 8) * _L, _L)] * a[k]
        return carry
      lax.fori_loop(0, sz // _L, sbody, 0)

      if j >= 1:
        scatters[j - 1].wait()
      scatters.append(pltpu.async_copy(
          sbuf.at[pl.ds(0, sz)], z_sh.at[dst_v.at[pl.ds(off, sz)]], ssem,
          add=True))
      off += sz

    scatters[-1].wait()
    plsc.subcore_barrier()
    if base == 0:
      pltpu.sync_copy(z_sh.at[pl.ds(r0, _RPT)],
                      zpart_hbm.at[c, pl.ds(r0, _RPT)])
    else:
      pltpu.sync_copy(z_sh.at[pl.ds(r0, _RPT)], zb_v)
      def dbody(g, carry):
        dv = d0_v[pl.ds(g * _L, _L)] + d1_v[pl.ds(g * _L, _L)]
        w = jnp.where(dv > 0.0, 1.0 / dv, 0.0)
        for k in range(_L):
          r = g * _L + k
          zb_v[r, :] = zb_v[r, :] * w[k]
        return carry
      lax.fori_loop(0, _RPT // _L, dbody, 0)
      pltpu.sync_copy(zb_v, zpart_hbm.at[c, pl.ds(r0, _RPT)])

  return functools.partial(
      pl.kernel,
      out_type=jax.ShapeDtypeStruct((_NC, _NP, _DE), jnp.float32),
      mesh=_MESH,
      compiler_params=_SC_PARAMS,
      scratch_types=[
          pltpu.VMEM((cnt,), jnp.int32),
          pltpu.VMEM((cnt,), jnp.float32),
          pltpu.VMEM((_RMAX // 8, 128), jnp.float32),
          pltpu.VMEM((_RMAX // 8, 128), jnp.float32),
          pltpu.VMEM((_RMAX, _DE), jnp.float32),
          pltpu.VMEM((_RMAX, _DE), jnp.float32),
          pltpu.VMEM((_RPT, _DE), jnp.float32),
          pltpu.VMEM((_RPT,), jnp.float32),
          pltpu.VMEM((_RPT,), jnp.float32),
          pltpu.VMEM_SHARED((_NP, _DE), jnp.float32),
          pltpu.SemaphoreType.DMA,
          pltpu.SemaphoreType.DMA,
      ],
  )(_sc_zsum)


_sc_zsum_calls = (_make_zsum(0, _SPLIT), _make_zsum(_SPLIT, _E - _SPLIT))


def _s_tc(x_ref, a_ref, b_ref, o_ref):
  o_ref[...] = lax.dot_general(a_ref[...], x_ref[...], (((1,), (1,)), ((), ())),
                               preferred_element_type=jnp.float32) + b_ref[...]


def _out_tc(x_ref, za_ref, zb_ref, w_ref, b_ref, o_ref):
  nb = x_ref.shape[0]
  z = za_ref[...] + zb_ref[...]
  dn = (((1,), (1,)), ((), ()))
  acc = lax.dot_general(x_ref[...], w_ref[:, :_DIN], dn,
                        preferred_element_type=jnp.float32)
  acc += lax.dot_general(z, w_ref[:, _DIN:], dn,
                         preferred_element_type=jnp.float32)
  o_ref[...] = jnp.maximum(acc + b_ref[...], 0.0).reshape(nb, 1, _DOUT)


def kernel(nfeats, efeats, edge_index, W_w, W_b, A_w, A_b):
  ei = edge_index.astype(jnp.int32)
  ef0 = efeats[:_SPLIT].reshape(_SPLIT // 8, 128).astype(jnp.float32)
  ef1 = efeats[_SPLIT:].reshape((_E - _SPLIT) // 8, 128).astype(jnp.float32)
  X2d = nfeats.reshape(_N, _DIN)

  A8 = jnp.zeros((8, _DIN), jnp.float32)
  A8 = A8.at[0].set(A_w[0, :_DIN]).at[1].set(A_w[0, _DIN:])
  sp8 = pl.pallas_call(
      _s_tc,
      out_shape=jax.ShapeDtypeStruct((8, _N), jnp.float32),
  )(X2d, A8, jnp.zeros((8, 1), jnp.float32).at[1, 0].set(A_b[0]))

  eexp, dpart, dstl = _sc_denom_call(sp8, ei)
  zpart0 = _sc_zsum_calls[0](dstl, eexp, ef0)
  zpart = _sc_zsum_calls[1](dstl, eexp, ef1, zpart0, dpart)

  out = pl.pallas_call(
      _out_tc,
      grid=(10,),
      in_specs=[
          pl.BlockSpec((_N // 10, _DIN), lambda i: (i, 0)),
          pl.BlockSpec((_N // 10, _DE), lambda i: (i, 0)),
          pl.BlockSpec((_N // 10, _DE), lambda i: (i, 0)),
          pl.BlockSpec((_DOUT, _DIN + _DE), lambda i: (0, 0)),
          pl.BlockSpec((1, _DOUT), lambda i: (0, 0)),
      ],
      out_specs=pl.BlockSpec((_N // 10, 1, _DOUT), lambda i: (i, 0, 0)),
      out_shape=jax.ShapeDtypeStruct((_N, 1, _DOUT), jnp.float32),
  )(X2d, zpart[0, :_N], zpart[1, :_N],
    W_w, W_b.reshape(1, _DOUT))
  return out
